# baseline (speedup 1.0000x reference)
import functools

import jax
import jax.numpy as jnp
from jax import lax
from jax.experimental import pallas as pl
from jax.experimental.pallas import tpu as pltpu
from jax.experimental.pallas import tpu_sc as plsc

B, L, V, D = 16384, 20, 1000000, 64
H1, H2 = 128, 64

NC, NS = 2, 16
NW = NC * NS
BAGS_W = B // NW
CB = 32
NCH = BAGS_W // CB
ROWS_CH = CB * L
IG = 128
NG = ROWS_CH // IG

_mesh = plsc.VectorSubcoreMesh(core_axis_name="c", subcore_axis_name="s")


@functools.partial(
    pl.kernel,
    out_type=(
        jax.ShapeDtypeStruct((B, D), jnp.float32),
        jax.ShapeDtypeStruct((B, D), jnp.float32),
    ),
    mesh=_mesh,
    scratch_types=[
        pltpu.VMEM((ROWS_CH,), jnp.int32),
        pltpu.VMEM((ROWS_CH, D), jnp.float32),
        pltpu.VMEM((CB, D), jnp.float32),
        pltpu.SemaphoreType.DMA,
    ],
    compiler_params=pltpu.CompilerParams(use_tc_tiling_on_sc=False),
)
def _pool_kernel(qidx_hbm, cidx_hbm, tq_hbm, tc_hbm, qout_hbm, cout_hbm,
                 idx_v, rows_v, out_v, sem):
    wid = lax.axis_index("s") * NC + lax.axis_index("c")

    for idx_hbm, tab_hbm, out_hbm in (
        (qidx_hbm, tq_hbm, qout_hbm),
        (cidx_hbm, tc_hbm, cout_hbm),
    ):
        def chunk_body(c, _, idx_hbm=idx_hbm, tab_hbm=tab_hbm, out_hbm=out_hbm):
            bag0 = wid * BAGS_W + c * CB
            pltpu.sync_copy(idx_hbm.at[pl.ds(bag0 * L, ROWS_CH)], idx_v)
            cps = [
                pltpu.async_copy(
                    tab_hbm.at[idx_v.at[pl.ds(j * IG, IG)]],
                    rows_v.at[pl.ds(j * IG, IG)],
                    sem,
                )
                for j in range(NG)
            ]
            for cp in cps:
                cp.wait()

            def bag_body(b, _):
                r0 = b * L
                for d in range(D // 16):
                    sl = pl.ds(d * 16, 16)
                    acc = rows_v[r0, sl]
                    for l in range(1, L):
                        acc = acc + rows_v[r0 + l, sl]
                    out_v[b, sl] = acc
                return 0

            lax.fori_loop(0, CB, bag_body, 0, unroll=False)
            pltpu.sync_copy(out_v, out_hbm.at[pl.ds(bag0, CB)])
            return 0

        lax.fori_loop(0, NCH, chunk_body, 0, unroll=False)


BM = 1024


def _mlp_body(qx, cx, qw1, qb1, qw2, qb2, cw1, cb1, cw2, cb2, qo, co):
    qh = jnp.maximum(
        jnp.dot(qx[...], qw1[...], preferred_element_type=jnp.float32) + qb1[...], 0.0)
    qo[...] = jnp.maximum(
        jnp.dot(qh, qw2[...], preferred_element_type=jnp.float32) + qb2[...], 0.0)
    ch = jnp.maximum(
        jnp.dot(cx[...], cw1[...], preferred_element_type=jnp.float32) + cb1[...], 0.0)
    co[...] = jnp.maximum(
        jnp.dot(ch, cw2[...], preferred_element_type=jnp.float32) + cb2[...], 0.0)


_mlp_call = pl.pallas_call(
    _mlp_body,
    grid=(B // BM,),
    in_specs=[
        pl.BlockSpec((BM, D), lambda i: (i, 0)),
        pl.BlockSpec((BM, D), lambda i: (i, 0)),
        pl.BlockSpec((D, H1), lambda i: (0, 0)),
        pl.BlockSpec((1, H1), lambda i: (0, 0)),
        pl.BlockSpec((H1, H2), lambda i: (0, 0)),
        pl.BlockSpec((1, H2), lambda i: (0, 0)),
        pl.BlockSpec((D, H1), lambda i: (0, 0)),
        pl.BlockSpec((1, H1), lambda i: (0, 0)),
        pl.BlockSpec((H1, H2), lambda i: (0, 0)),
        pl.BlockSpec((1, H2), lambda i: (0, 0)),
    ],
    out_specs=[
        pl.BlockSpec((BM, H2), lambda i: (i, 0)),
        pl.BlockSpec((BM, H2), lambda i: (i, 0)),
    ],
    out_shape=[
        jax.ShapeDtypeStruct((B, H2), jnp.float32),
        jax.ShapeDtypeStruct((B, H2), jnp.float32),
    ],
)


def kernel(query_indices, candidate_indices, table_query, table_candidate,
           q_w1, q_b1, q_w2, q_b2, c_w1, c_b1, c_w2, c_b2):
    qidx = query_indices.reshape(B * L).astype(jnp.int32)
    cidx = candidate_indices.reshape(B * L).astype(jnp.int32)
    q_pooled, c_pooled = _pool_kernel(qidx, cidx, table_query, table_candidate)
    qe, ce = _mlp_call(
        q_pooled, c_pooled,
        q_w1, q_b1.reshape(1, H1), q_w2, q_b2.reshape(1, H2),
        c_w1, c_b1.reshape(1, H1), c_w2, c_b2.reshape(1, H2),
    )
    return (qe, ce)


# speedup vs baseline: 1.0886x; 1.0886x over previous
import functools

import jax
import jax.numpy as jnp
from jax import lax
from jax.experimental import pallas as pl
from jax.experimental.pallas import tpu as pltpu
from jax.experimental.pallas import tpu_sc as plsc

B, L, V, D = 16384, 20, 1000000, 64
H1, H2 = 128, 64

NC, NS = 2, 16
NW = NC * NS
BAGS_W = B // NW
IG = 128
NG = BAGS_W // IG
PAIRS_W = BAGS_W // 2

_mesh = plsc.VectorSubcoreMesh(core_axis_name="c", subcore_axis_name="s")


def _fire(tab_hbm, idx_v, l, rows_v, sem):
    return [
        pltpu.async_copy(
            tab_hbm.at[idx_v.at[l, pl.ds(j * IG, IG)]],
            rows_v.at[pl.ds(j * IG, IG)],
            sem,
        )
        for j in range(NG)
    ]


@functools.partial(
    pl.kernel,
    out_type=(
        jax.ShapeDtypeStruct((B // 2, 2 * D), jnp.float32),
        jax.ShapeDtypeStruct((B // 2, 2 * D), jnp.float32),
    ),
    mesh=_mesh,
    scratch_types=[
        pltpu.VMEM((L, BAGS_W), jnp.int32),
        pltpu.VMEM((L, BAGS_W), jnp.int32),
        pltpu.VMEM((BAGS_W, D), jnp.float32),
        pltpu.VMEM((BAGS_W, D), jnp.float32),
        pltpu.VMEM((PAIRS_W, 2 * D), jnp.float32),
        pltpu.SemaphoreType.DMA,
        pltpu.SemaphoreType.DMA,
    ],
    compiler_params=pltpu.CompilerParams(use_tc_tiling_on_sc=False),
)
def _pool_kernel(qidx_hbm, cidx_hbm, tq_hbm, tc_hbm, qout_hbm, cout_hbm,
                 qidx_v, cidx_v, rows_a, rows_b, acc_v, sem_a, sem_b):
    wid = lax.axis_index("s") * NC + lax.axis_index("c")
    bag0 = wid * BAGS_W

    pltpu.sync_copy(qidx_hbm.at[:, pl.ds(bag0, BAGS_W)], qidx_v)
    pltpu.sync_copy(cidx_hbm.at[:, pl.ds(bag0, BAGS_W)], cidx_v)

    rows = (rows_a, rows_b)
    sems = (sem_a, sem_b)
    jobs = [(qidx_v, tq_hbm, qout_hbm, l) for l in range(L)] + \
           [(cidx_v, tc_hbm, cout_hbm, l) for l in range(L)]

    def zero_acc():
        z = jnp.zeros((16,), jnp.float32)

        def zbody(r, _):
            for c8 in range(8):
                acc_v[r, pl.ds(c8 * 16, 16)] = z
            return 0

        lax.fori_loop(0, PAIRS_W, zbody, 0, unroll=False)

    def flush_acc(out_hbm):
        pltpu.sync_copy(acc_v, out_hbm.at[pl.ds(wid * PAIRS_W, PAIRS_W)])

    def accumulate(rows_v):
        def abody(i2, _):
            for u in range(2):
                for d in range(4):
                    i = i2 * 2 + u
                    val = rows_v[i, pl.ds(d * 16, 16)]
                    sl = pl.ds(u * 64 + d * 16, 16)
                    acc_v[i2, sl] = acc_v[i2, sl] + val
            return 0

        lax.fori_loop(0, PAIRS_W, abody, 0, unroll=False)

    zero_acc()
    idx0, tab0, _, l0 = jobs[0]
    pend = _fire(tab0, idx0, l0, rows[0], sems[0])
    for j, (idx_v, tab_hbm, out_hbm, l) in enumerate(jobs):
        if j + 1 < len(jobs):
            nidx, ntab, _, nl = jobs[j + 1]
            nxt = _fire(ntab, nidx, nl, rows[(j + 1) % 2], sems[(j + 1) % 2])
        else:
            nxt = None
        for cp in pend:
            cp.wait()
        accumulate(rows[j % 2])
        if l == L - 1:
            flush_acc(out_hbm)
            if j + 1 < len(jobs):
                zero_acc()
        pend = nxt


BMP = 2048


def _mlp_body(qx, cx, qw1, qb1, qw2, qb2, cw1, cb1, cw2, cb2, qo, co):
    def tower(xp, w1, b1, w2, b2, out_ref):
        xe = xp[:, :D]
        xo = xp[:, D:]
        x = jnp.concatenate([xe, xo], axis=0)
        h = jnp.maximum(
            jnp.dot(x, w1[...], preferred_element_type=jnp.float32) + b1[...], 0.0)
        y = jnp.maximum(
            jnp.dot(h, w2[...], preferred_element_type=jnp.float32) + b2[...], 0.0)
        ye = y[:BMP]
        yo = y[BMP:]
        out_ref[...] = jnp.stack([ye, yo], axis=1).reshape(2 * BMP, H2)

    tower(qx[...], qw1, qb1, qw2, qb2, qo)
    tower(cx[...], cw1, cb1, cw2, cb2, co)


_mlp_call = pl.pallas_call(
    _mlp_body,
    grid=(B // (2 * BMP),),
    in_specs=[
        pl.BlockSpec((BMP, 2 * D), lambda i: (i, 0)),
        pl.BlockSpec((BMP, 2 * D), lambda i: (i, 0)),
        pl.BlockSpec((D, H1), lambda i: (0, 0)),
        pl.BlockSpec((1, H1), lambda i: (0, 0)),
        pl.BlockSpec((H1, H2), lambda i: (0, 0)),
        pl.BlockSpec((1, H2), lambda i: (0, 0)),
        pl.BlockSpec((D, H1), lambda i: (0, 0)),
        pl.BlockSpec((1, H1), lambda i: (0, 0)),
        pl.BlockSpec((H1, H2), lambda i: (0, 0)),
        pl.BlockSpec((1, H2), lambda i: (0, 0)),
    ],
    out_specs=[
        pl.BlockSpec((2 * BMP, H2), lambda i: (i, 0)),
        pl.BlockSpec((2 * BMP, H2), lambda i: (i, 0)),
    ],
    out_shape=[
        jax.ShapeDtypeStruct((B, H2), jnp.float32),
        jax.ShapeDtypeStruct((B, H2), jnp.float32),
    ],
)


def kernel(query_indices, candidate_indices, table_query, table_candidate,
           q_w1, q_b1, q_w2, q_b2, c_w1, c_b1, c_w2, c_b2):
    qidx = query_indices.T.astype(jnp.int32)
    cidx = candidate_indices.T.astype(jnp.int32)
    q_pooled, c_pooled = _pool_kernel(qidx, cidx, table_query, table_candidate)
    qe, ce = _mlp_call(
        q_pooled, c_pooled,
        q_w1, q_b1.reshape(1, H1), q_w2, q_b2.reshape(1, H2),
        c_w1, c_b1.reshape(1, H1), c_w2, c_b2.reshape(1, H2),
    )
    return (qe, ce)


# speedup vs baseline: 1.0895x; 1.0008x over previous
import functools

import jax
import jax.numpy as jnp
from jax import lax
from jax.experimental import pallas as pl
from jax.experimental.pallas import tpu as pltpu
from jax.experimental.pallas import tpu_sc as plsc

B, L, V, D = 16384, 20, 1000000, 64
H1, H2 = 128, 64

NC, NS = 2, 16
NW = NC * NS
BAGS_W = B // NW
IG = 128
NG = BAGS_W // IG
PAIRS_W = BAGS_W // 2
FB = 1024

_mesh = plsc.VectorSubcoreMesh(core_axis_name="c", subcore_axis_name="s")


def _fmt_body(qi, ci, qo, co):
    qo[...] = qi[...].reshape(L, FB // IG, IG)
    co[...] = ci[...].reshape(L, FB // IG, IG)


_fmt_call = pl.pallas_call(
    _fmt_body,
    grid=(B // FB,),
    in_specs=[
        pl.BlockSpec((L, FB), lambda i: (0, i)),
        pl.BlockSpec((L, FB), lambda i: (0, i)),
    ],
    out_specs=[
        pl.BlockSpec((L, FB // IG, IG), lambda i: (0, i, 0)),
        pl.BlockSpec((L, FB // IG, IG), lambda i: (0, i, 0)),
    ],
    out_shape=[
        jax.ShapeDtypeStruct((L, B // IG, IG), jnp.int32),
        jax.ShapeDtypeStruct((L, B // IG, IG), jnp.int32),
    ],
)


def _fire(tab_hbm, idx_v, l, rows_v, sem):
    return [
        pltpu.async_copy(
            tab_hbm.at[idx_v.at[l, j]],
            rows_v.at[pl.ds(j * IG, IG)],
            sem,
        )
        for j in range(NG)
    ]


@functools.partial(
    pl.kernel,
    out_type=(
        jax.ShapeDtypeStruct((B // 2, 2 * D), jnp.float32),
        jax.ShapeDtypeStruct((B // 2, 2 * D), jnp.float32),
    ),
    mesh=_mesh,
    scratch_types=[
        pltpu.VMEM((L, NG, IG), jnp.int32),
        pltpu.VMEM((L, NG, IG), jnp.int32),
        pltpu.VMEM((BAGS_W, D), jnp.float32),
        pltpu.VMEM((BAGS_W, D), jnp.float32),
        pltpu.VMEM((PAIRS_W, 2 * D), jnp.float32),
        pltpu.SemaphoreType.DMA,
        pltpu.SemaphoreType.DMA,
    ],
    compiler_params=pltpu.CompilerParams(use_tc_tiling_on_sc=False),
)
def _pool_kernel(qidx_hbm, cidx_hbm, tq_hbm, tc_hbm, qout_hbm, cout_hbm,
                 qidx_v, cidx_v, rows_a, rows_b, acc_v, sem_a, sem_b):
    wid = lax.axis_index("s") * NC + lax.axis_index("c")

    pltpu.sync_copy(qidx_hbm.at[:, pl.ds(wid * NG, NG), :], qidx_v)
    pltpu.sync_copy(cidx_hbm.at[:, pl.ds(wid * NG, NG), :], cidx_v)

    rows = (rows_a, rows_b)
    sems = (sem_a, sem_b)
    jobs = [(qidx_v, tq_hbm, qout_hbm, l) for l in range(L)] + \
           [(cidx_v, tc_hbm, cout_hbm, l) for l in range(L)]

    def zero_acc():
        z = jnp.zeros((16,), jnp.float32)

        def zbody(r, _):
            for c8 in range(8):
                acc_v[r, pl.ds(c8 * 16, 16)] = z
            return 0

        lax.fori_loop(0, PAIRS_W, zbody, 0, unroll=False)

    def flush_acc(out_hbm):
        pltpu.sync_copy(acc_v, out_hbm.at[pl.ds(wid * PAIRS_W, PAIRS_W)])

    def accumulate(rows_v):
        def abody(i2, _):
            for u in range(2):
                for d in range(4):
                    i = i2 * 2 + u
                    val = rows_v[i, pl.ds(d * 16, 16)]
                    sl = pl.ds(u * 64 + d * 16, 16)
                    acc_v[i2, sl] = acc_v[i2, sl] + val
            return 0

        lax.fori_loop(0, PAIRS_W, abody, 0, unroll=False)

    zero_acc()
    idx0, tab0, _, l0 = jobs[0]
    pend = _fire(tab0, idx0, l0, rows[0], sems[0])
    for j, (idx_v, tab_hbm, out_hbm, l) in enumerate(jobs):
        if j + 1 < len(jobs):
            nidx, ntab, _, nl = jobs[j + 1]
            nxt = _fire(ntab, nidx, nl, rows[(j + 1) % 2], sems[(j + 1) % 2])
        else:
            nxt = None
        for cp in pend:
            cp.wait()
        accumulate(rows[j % 2])
        if l == L - 1:
            flush_acc(out_hbm)
            if j + 1 < len(jobs):
                zero_acc()
        pend = nxt


BMP = 2048


def _mlp_body(qx, cx, qw1, qb1, qw2, qb2, cw1, cb1, cw2, cb2, qo, co):
    def tower(xp, w1, b1, w2, b2, out_ref):
        xe = xp[:, :D]
        xo = xp[:, D:]
        x = jnp.concatenate([xe, xo], axis=0)
        h = jnp.maximum(
            jnp.dot(x, w1[...], preferred_element_type=jnp.float32) + b1[...], 0.0)
        y = jnp.maximum(
            jnp.dot(h, w2[...], preferred_element_type=jnp.float32) + b2[...], 0.0)
        ye = y[:BMP]
        yo = y[BMP:]
        out_ref[...] = jnp.stack([ye, yo], axis=1).reshape(2 * BMP, H2)

    tower(qx[...], qw1, qb1, qw2, qb2, qo)
    tower(cx[...], cw1, cb1, cw2, cb2, co)


_mlp_call = pl.pallas_call(
    _mlp_body,
    grid=(B // (2 * BMP),),
    in_specs=[
        pl.BlockSpec((BMP, 2 * D), lambda i: (i, 0)),
        pl.BlockSpec((BMP, 2 * D), lambda i: (i, 0)),
        pl.BlockSpec((D, H1), lambda i: (0, 0)),
        pl.BlockSpec((1, H1), lambda i: (0, 0)),
        pl.BlockSpec((H1, H2), lambda i: (0, 0)),
        pl.BlockSpec((1, H2), lambda i: (0, 0)),
        pl.BlockSpec((D, H1), lambda i: (0, 0)),
        pl.BlockSpec((1, H1), lambda i: (0, 0)),
        pl.BlockSpec((H1, H2), lambda i: (0, 0)),
        pl.BlockSpec((1, H2), lambda i: (0, 0)),
    ],
    out_specs=[
        pl.BlockSpec((2 * BMP, H2), lambda i: (i, 0)),
        pl.BlockSpec((2 * BMP, H2), lambda i: (i, 0)),
    ],
    out_shape=[
        jax.ShapeDtypeStruct((B, H2), jnp.float32),
        jax.ShapeDtypeStruct((B, H2), jnp.float32),
    ],
)


def kernel(query_indices, candidate_indices, table_query, table_candidate,
           q_w1, q_b1, q_w2, q_b2, c_w1, c_b1, c_w2, c_b2):
    qidx, cidx = _fmt_call(query_indices.T.astype(jnp.int32),
                           candidate_indices.T.astype(jnp.int32))
    q_pooled, c_pooled = _pool_kernel(qidx, cidx, table_query, table_candidate)
    qe, ce = _mlp_call(
        q_pooled, c_pooled,
        q_w1, q_b1.reshape(1, H1), q_w2, q_b2.reshape(1, H2),
        c_w1, c_b1.reshape(1, H1), c_w2, c_b2.reshape(1, H2),
    )
    return (qe, ce)
